# baseline (speedup 1.0000x reference)
import functools
import jax
import jax.numpy as jnp
from jax import lax
from jax.experimental import pallas as pl
from jax.experimental.pallas import tpu as pltpu
from jax.experimental.pallas import tpu_sc as plsc

NC = 2
NS = 16
L = 16

B = 4096
N = 20
D = 32
P = N * (N - 1) // 2
OUT_ROW = (N + P) * D
NW = NC * NS
RPW = B // NW
R = 4
NCHUNK = RPW // R
CHUNK_IDX = R * N
CHUNK_OUT = R * OUT_ROW


def _body(uidx_hbm, midx_hbm, utab_hbm, itab_hbm, w_hbm, out_hbm,
          uidx_v, midx_v, urows_v, w_v, rows0_v, rows1_v, out0_v, out1_v,
          sem_g0, sem_g1, sem_o0, sem_o1):
    wid = lax.axis_index("s") * NC + lax.axis_index("c")
    row_base = wid * RPW

    pltpu.sync_copy(uidx_hbm.at[pl.ds(row_base, RPW)], uidx_v)
    pltpu.sync_copy(midx_hbm.at[pl.ds(row_base * N, RPW * N)], midx_v)
    pltpu.sync_copy(w_hbm, w_v)

    pltpu.async_copy(utab_hbm.at[uidx_v], urows_v, sem_g0).wait()

    rows_bufs = (rows0_v, rows1_v)
    out_bufs = (out0_v, out1_v)
    sem_g = (sem_g0, sem_g1)
    sem_o = (sem_o0, sem_o1)

    def gather_chunk(c, buf):
        idx = midx_v.at[pl.ds(c * CHUNK_IDX, CHUNK_IDX)]
        return pltpu.async_copy(itab_hbm.at[idx], rows_bufs[buf], sem_g[buf])

    gather_chunk(0, 0)

    def compute_chunk(c, buf):
        rows_v = rows_bufs[buf]
        out_v = out_bufs[buf]

        def row_body(r, _):
            crow = c * R + r
            obase = r * OUT_ROW
            u0 = urows_v[crow, pl.ds(0, L)]
            u1 = urows_v[crow, pl.ds(L, L)]
            v0 = []
            v1 = []
            for i in range(N):
                e0 = rows_v[r * N + i, pl.ds(0, L)]
                e1 = rows_v[r * N + i, pl.ds(L, L)]
                a0 = e0 * w_v[i, pl.ds(0, L)]
                a1 = e1 * w_v[i, pl.ds(L, L)]
                v0.append(a0)
                v1.append(a1)
                out_v[pl.ds(obase + D * i, L)] = u0 * a0
                out_v[pl.ds(obase + D * i + L, L)] = u1 * a1
            off = N * D
            for i in range(N):
                for j in range(i + 1, N):
                    out_v[pl.ds(obase + off, L)] = v0[i] * v0[j]
                    out_v[pl.ds(obase + off + L, L)] = v1[i] * v1[j]
                    off += D
            return _

        lax.fori_loop(0, R, row_body, None)

    def outer(i, _):
        for b in range(2):
            c = 2 * i + b
            pltpu.make_async_copy(
                itab_hbm.at[midx_v.at[pl.ds(0, CHUNK_IDX)]],
                rows_bufs[b], sem_g[b]).wait()
            gather_chunk(lax.rem(c + 1, NCHUNK), 1 - b)
            @pl.when(i >= 1)
            def _wait_out():
                pltpu.make_async_copy(
                    out_bufs[b], out_hbm.at[pl.ds(0, CHUNK_OUT)],
                    sem_o[b]).wait()
            compute_chunk(c, b)
            dst = out_hbm.at[pl.ds((row_base + c * R) * OUT_ROW, CHUNK_OUT)]
            pltpu.async_copy(out_bufs[b], dst, sem_o[b])
        return _

    lax.fori_loop(0, NCHUNK // 2, outer, None)

    pltpu.make_async_copy(
        itab_hbm.at[midx_v.at[pl.ds(0, CHUNK_IDX)]], rows_bufs[0],
        sem_g[0]).wait()
    for b in range(2):
        pltpu.make_async_copy(
            out_bufs[b], out_hbm.at[pl.ds(0, CHUNK_OUT)], sem_o[b]).wait()


def kernel(user, memory, user_table, item_table, weights):
    uidx = user.reshape(-1).astype(jnp.int32)
    midx = memory.reshape(-1).astype(jnp.int32)
    w2d = jnp.broadcast_to(
        weights.astype(jnp.float32)[:, None], (N, D))

    mesh = plsc.VectorSubcoreMesh(core_axis_name="c", subcore_axis_name="s")
    k = pl.kernel(
        _body,
        out_type=jax.ShapeDtypeStruct((B * OUT_ROW,), jnp.float32),
        mesh=mesh,
        compiler_params=pltpu.CompilerParams(use_tc_tiling_on_sc=False),
        scratch_types=[
            pltpu.VMEM((RPW,), jnp.int32),
            pltpu.VMEM((RPW * N,), jnp.int32),
            pltpu.VMEM((RPW, D), jnp.float32),
            pltpu.VMEM((N, D), jnp.float32),
            pltpu.VMEM((CHUNK_IDX, D), jnp.float32),
            pltpu.VMEM((CHUNK_IDX, D), jnp.float32),
            pltpu.VMEM((CHUNK_OUT,), jnp.float32),
            pltpu.VMEM((CHUNK_OUT,), jnp.float32),
            pltpu.SemaphoreType.DMA,
            pltpu.SemaphoreType.DMA,
            pltpu.SemaphoreType.DMA,
            pltpu.SemaphoreType.DMA,
        ],
    )
    out = k(uidx, midx,
            user_table.astype(jnp.float32),
            item_table.astype(jnp.float32),
            w2d)
    return out.reshape(B, OUT_ROW)
